# baseline (speedup 1.0000x reference)
import functools

import jax
import jax.numpy as jnp
from jax import lax
from jax.experimental import pallas as pl
from jax.experimental.pallas import tpu as pltpu

_TINY = float(jnp.finfo(jnp.float32).tiny)
_KS0 = 0
_KS1 = 42
_KS2 = 0x1BD11BDA ^ 0 ^ 42
_ROT = ((13, 15, 26, 6), (17, 29, 16, 24))


def _i32(v):
    return jnp.int32(jnp.uint32(v))


def _threefry_bits(x1):
    ks = (_i32(_KS0), _i32(_KS1), _i32(_KS2))
    x0 = jnp.zeros_like(x1) + ks[0]
    x1 = x1 + ks[1]
    for i in range(5):
        for r in _ROT[i % 2]:
            x0 = x0 + x1
            x1 = (lax.shift_left(x1, jnp.int32(r))
                  | lax.shift_right_logical(x1, jnp.int32(32 - r)))
            x1 = lax.bitwise_xor(x0, x1)
        x0 = x0 + ks[(i + 1) % 3]
        x1 = x1 + ks[(i + 2) % 3] + _i32(i + 1)
    return lax.bitwise_xor(x0, x1)


def _score(v, f):
    bits = _threefry_bits(f)
    fb = lax.bitwise_or(lax.shift_right_logical(bits, jnp.int32(9)),
                        jnp.int32(0x3F800000))
    u = lax.bitcast_convert_type(fb, jnp.float32) - jnp.float32(1.0)
    u = jnp.maximum(u, jnp.float32(_TINY))
    return v / (-jnp.log(u))


def _mc_pool_kernel(xe_ref, xo_ref, o_ref, *, cb, h, w):
    ch0 = pl.program_id(0) * cb
    ph, pw = h // 2, w // 2
    ve = xe_ref[:, :, 0, 0, :]
    vo = xo_ref[:, :, 0, 0, :]

    ch = (lax.broadcasted_iota(jnp.int32, (cb, ph, w), 0) + ch0) * _i32(h * w)
    i = lax.broadcasted_iota(jnp.int32, (cb, ph, w), 1)
    c = lax.broadcasted_iota(jnp.int32, (cb, ph, w), 2)
    f = (ch + i * _i32(4 * pw) + c * _i32(2) - lax.bitwise_and(c, jnp.int32(1)))

    s0 = _score(ve, f)
    s1 = _score(vo, f + _i32(2))
    s0r = pltpu.roll(s0, w - 1, 2)
    s1r = pltpu.roll(s1, w - 1, 2)

    best = s0
    idx = jnp.zeros_like(s0)
    idx = jnp.where(s0r > best, jnp.float32(1.0), idx)
    best = jnp.maximum(best, s0r)
    idx = jnp.where(s1 > best, jnp.float32(2.0), idx)
    best = jnp.maximum(best, s1)
    idx = jnp.where(s1r > best, jnp.float32(3.0), idx)

    sel = (lax.broadcasted_iota(jnp.int32, (w, pw), 0)
           == lax.broadcasted_iota(jnp.int32, (w, pw), 1) * 2
           ).astype(jnp.float32)
    for b in range(cb):
        o_ref[b] = jnp.dot(idx[b], sel)


def kernel(x):
    batch, chan, h, w = x.shape
    nch = batch * chan
    cb = 4
    xr = x.reshape(nch, h // 2, 2, 1, w)
    grid = (nch // cb,)
    out = pl.pallas_call(
        functools.partial(_mc_pool_kernel, cb=cb, h=h, w=w),
        grid=grid,
        in_specs=[
            pl.BlockSpec((cb, h // 2, 1, 1, w), lambda i: (i, 0, 0, 0, 0)),
            pl.BlockSpec((cb, h // 2, 1, 1, w), lambda i: (i, 0, 1, 0, 0)),
        ],
        out_specs=pl.BlockSpec((cb, h // 2, w // 2), lambda i: (i, 0, 0)),
        out_shape=jax.ShapeDtypeStruct((nch, h // 2, w // 2), jnp.float32),
        compiler_params=pltpu.CompilerParams(
            dimension_semantics=("parallel",)),
    )(xr, xr)
    return out.reshape(batch, chan, h // 2, w // 2)


# speedup vs baseline: 1.7667x; 1.7667x over previous
import functools

import jax
import jax.numpy as jnp
from jax import lax
from jax.experimental import pallas as pl
from jax.experimental.pallas import tpu as pltpu

_TINY = float(jnp.finfo(jnp.float32).tiny)
_KS1 = 42
_KS2 = 0x1BD11BDA ^ 0 ^ 42
_ROT = ((13, 15, 26, 6), (17, 29, 16, 24))


def _i32(v):
    return jnp.int32(jnp.uint32(v))


def _rotl(x, r):
    return (lax.shift_left(x, jnp.int32(r))
            | lax.shift_right_logical(x, jnp.int32(32 - r)))


def _threefry_bits(f):
    ks = (_i32(0), _i32(_KS1), _i32(_KS2))
    x1 = f + ks[1]
    x0 = x1
    x1 = lax.bitwise_xor(_rotl(x1, _ROT[0][0]), x0)
    for r in _ROT[0][1:]:
        x0 = x0 + x1
        x1 = lax.bitwise_xor(_rotl(x1, r), x0)
    x0 = x0 + ks[1]
    x1 = x1 + ks[2] + _i32(1)
    for i in range(1, 5):
        for r in _ROT[i % 2]:
            x0 = x0 + x1
            x1 = lax.bitwise_xor(_rotl(x1, r), x0)
        x0 = x0 + ks[(i + 1) % 3]
        x1 = x1 + ks[(i + 2) % 3] + _i32(i + 1)
    return lax.bitwise_xor(x0, x1)


def _score(v, f):
    bits = _threefry_bits(f)
    fb = lax.bitwise_or(lax.shift_right_logical(bits, jnp.int32(9)),
                        jnp.int32(0x3F800000))
    u = lax.bitcast_convert_type(fb, jnp.float32) - jnp.float32(1.0)
    u = jnp.maximum(u, jnp.float32(_TINY))
    return v / (-jnp.log(u))


def _mc_pool_kernel(x_ref, o_ref, *, cb, h, w):
    ch0 = pl.program_id(0) * cb
    ph, pw = h // 2, w // 2
    v = x_ref[...]

    ch = (lax.broadcasted_iota(jnp.int32, (cb, h, w), 0) + ch0) * _i32(h * w)
    r = lax.broadcasted_iota(jnp.int32, (cb, h, w), 1)
    c = lax.broadcasted_iota(jnp.int32, (cb, h, w), 2)
    f = (ch + r * _i32(2 * pw)
         - lax.bitwise_and(r, jnp.int32(1)) * _i32(2 * (pw - 1))
         + c * _i32(2) - lax.bitwise_and(c, jnp.int32(1)))

    s = _score(v, f)
    sd = pltpu.roll(s, h - 1, 1)
    sr = pltpu.roll(s, w - 1, 2)
    sdr = pltpu.roll(sd, w - 1, 2)

    best = s
    idx = jnp.zeros_like(s)
    idx = jnp.where(sr > best, jnp.float32(1.0), idx)
    best = jnp.maximum(best, sr)
    idx = jnp.where(sd > best, jnp.float32(2.0), idx)
    best = jnp.maximum(best, sd)
    idx = jnp.where(sdr > best, jnp.float32(3.0), idx)

    csel = (lax.broadcasted_iota(jnp.int32, (w, pw), 0)
            == lax.broadcasted_iota(jnp.int32, (w, pw), 1) * 2
            ).astype(jnp.float32)
    rsel = (lax.broadcasted_iota(jnp.int32, (ph, h), 1)
            == lax.broadcasted_iota(jnp.int32, (ph, h), 0) * 2
            ).astype(jnp.float32)
    for b in range(cb):
        o_ref[b] = jnp.dot(rsel, jnp.dot(idx[b], csel))


def kernel(x):
    batch, chan, h, w = x.shape
    nch = batch * chan
    cb = 4
    xr = x.reshape(nch, h, w)
    grid = (nch // cb,)
    out = pl.pallas_call(
        functools.partial(_mc_pool_kernel, cb=cb, h=h, w=w),
        grid=grid,
        in_specs=[pl.BlockSpec((cb, h, w), lambda i: (i, 0, 0))],
        out_specs=pl.BlockSpec((cb, h // 2, w // 2), lambda i: (i, 0, 0)),
        out_shape=jax.ShapeDtypeStruct((nch, h // 2, w // 2), jnp.float32),
        compiler_params=pltpu.CompilerParams(
            dimension_semantics=("parallel",)),
    )(xr)
    return out.reshape(batch, chan, h // 2, w // 2)


# speedup vs baseline: 2.0239x; 1.1456x over previous
import functools

import jax
import jax.numpy as jnp
from jax import lax
from jax.experimental import pallas as pl
from jax.experimental.pallas import tpu as pltpu
from jax.experimental.pallas import tpu_sc as plsc

_TINY = float(jnp.finfo(jnp.float32).tiny)
_KS1 = 42
_KS2 = 0x1BD11BDA ^ 0 ^ 42
_ROT = ((13, 15, 26, 6), (17, 29, 16, 24))

_SC_CH = 96
_SC_R = 48


def _i32(v):
    return jnp.int32(jnp.uint32(v))


def _rotl(x, r):
    return (lax.shift_left(x, jnp.int32(r))
            | lax.shift_right_logical(x, jnp.int32(32 - r)))


def _threefry_bits(f):
    ks = (_i32(0), _i32(_KS1), _i32(_KS2))
    x1 = f + ks[1]
    x0 = x1
    x1 = lax.bitwise_xor(_rotl(x1, _ROT[0][0]), x0)
    for r in _ROT[0][1:]:
        x0 = x0 + x1
        x1 = lax.bitwise_xor(_rotl(x1, r), x0)
    x0 = x0 + ks[1]
    x1 = x1 + ks[2] + _i32(1)
    for i in range(1, 5):
        for r in _ROT[i % 2]:
            x0 = x0 + x1
            x1 = lax.bitwise_xor(_rotl(x1, r), x0)
        x0 = x0 + ks[(i + 1) % 3]
        x1 = x1 + ks[(i + 2) % 3] + _i32(i + 1)
    return lax.bitwise_xor(x0, x1)


def _uniform(f):
    bits = _threefry_bits(f)
    fb = lax.bitwise_or(lax.shift_right_logical(bits, jnp.int32(9)),
                        jnp.int32(0x3F800000))
    u = lax.bitcast_convert_type(fb, jnp.float32) - jnp.float32(1.0)
    return jnp.maximum(u, jnp.float32(_TINY))


def _score_tc(v, f):
    return v / (-jnp.log(_uniform(f)))


def _mc_pool_tc(x_ref, o_ref, *, cb, h, w):
    ch0 = pl.program_id(0) * cb
    ph, pw = h // 2, w // 2
    v = x_ref[...]

    ch = (lax.broadcasted_iota(jnp.int32, (cb, h, w), 0) + ch0) * _i32(h * w)
    r = lax.broadcasted_iota(jnp.int32, (cb, h, w), 1)
    c = lax.broadcasted_iota(jnp.int32, (cb, h, w), 2)
    f = (ch + r * _i32(2 * pw)
         - lax.bitwise_and(r, jnp.int32(1)) * _i32(2 * (pw - 1))
         + c * _i32(2) - lax.bitwise_and(c, jnp.int32(1)))

    s = _score_tc(v, f)
    sd = pltpu.roll(s, h - 1, 1)
    sr = pltpu.roll(s, w - 1, 2)
    sdr = pltpu.roll(sd, w - 1, 2)

    best = s
    idx = jnp.zeros_like(s)
    idx = jnp.where(sr > best, jnp.float32(1.0), idx)
    best = jnp.maximum(best, sr)
    idx = jnp.where(sd > best, jnp.float32(2.0), idx)
    best = jnp.maximum(best, sd)
    idx = jnp.where(sdr > best, jnp.float32(3.0), idx)

    csel = (lax.broadcasted_iota(jnp.int32, (w, pw), 0)
            == lax.broadcasted_iota(jnp.int32, (w, pw), 1) * 2
            ).astype(jnp.float32)
    rsel = (lax.broadcasted_iota(jnp.int32, (ph, h), 1)
            == lax.broadcasted_iota(jnp.int32, (ph, h), 0) * 2
            ).astype(jnp.float32)
    for b in range(cb):
        o_ref[b] = jnp.dot(rsel, jnp.dot(idx[b], csel))


def _neglog_sc(u):
    ub = lax.bitcast_convert_type(u, jnp.int32)
    e2 = lax.shift_right_logical(ub, jnp.int32(23)) - jnp.int32(127)
    m = lax.bitcast_convert_type(
        lax.bitwise_or(lax.bitwise_and(ub, jnp.int32(0x007FFFFF)),
                       jnp.int32(0x3F800000)), jnp.float32)
    big = m > jnp.float32(1.41421356)
    m = jnp.where(big, m * jnp.float32(0.5), m)
    e2 = jnp.where(big, e2 + jnp.int32(1), e2).astype(jnp.float32)
    x = m - jnp.float32(1.0)
    z = x * x
    p = jnp.float32(7.0376836292e-2)
    for c in (-1.1514610310e-1, 1.1676998740e-1, -1.2420140846e-1,
              1.4249322787e-1, -1.6668057665e-1, 2.0000714765e-1,
              -2.4999993993e-1, 3.3333331174e-1):
        p = p * x + jnp.float32(c)
    y = x * z * p
    y = y + e2 * jnp.float32(-2.12194440e-4)
    y = y - jnp.float32(0.5) * z
    r = x + y
    r = r + e2 * jnp.float32(0.693359375)
    return -r


def _mc_pool_sc(x_hbm, o_hbm, inbuf, outbuf, *, ch_start, h, w):
    ph, pw = h // 2, w // 2
    nw = 32
    cpw = _SC_CH // nw
    nchunk = ph // _SC_R
    wid = lax.axis_index("s") * jnp.int32(2) + lax.axis_index("c")
    ch_lo = wid * jnp.int32(cpw)
    ii = lax.iota(jnp.int32, 16)

    def unit(u, carry):
        chl = ch_lo + u // jnp.int32(nchunk)
        chunk = u % jnp.int32(nchunk)
        pltpu.sync_copy(
            x_hbm.at[jnp.int32(ch_start) + chl,
                     pl.ds(chunk * jnp.int32(2 * _SC_R), 2 * _SC_R)],
            inbuf)

        def row(i_out, carry_r):
            r0 = i_out * jnp.int32(2)
            rv0 = jnp.broadcast_to(r0, (16,))
            rv1 = rv0 + jnp.int32(1)
            frow = ((jnp.int32(ch_start) + chl) * jnp.int32(h * w)
                    + (chunk * jnp.int32(_SC_R) + i_out) * jnp.int32(4 * pw))

            def jvec(jv, carry_j):
                ce = jv * jnp.int32(32) + ii * jnp.int32(2)
                co = ce + jnp.int32(1)
                a = plsc.load_gather(inbuf, [rv0, ce])
                b = plsc.load_gather(inbuf, [rv0, co])
                cc = plsc.load_gather(inbuf, [rv1, ce])
                d = plsc.load_gather(inbuf, [rv1, co])
                fv = frow + jv * jnp.int32(64) + ii * jnp.int32(4)
                sa = a / _neglog_sc(_uniform(fv))
                sb = b / _neglog_sc(_uniform(fv + jnp.int32(1)))
                sc_ = cc / _neglog_sc(_uniform(fv + jnp.int32(2)))
                sd_ = d / _neglog_sc(_uniform(fv + jnp.int32(3)))
                best = sa
                idx = jnp.zeros_like(sa)
                idx = jnp.where(sb > best, jnp.float32(1.0), idx)
                best = jnp.maximum(best, sb)
                idx = jnp.where(sc_ > best, jnp.float32(2.0), idx)
                best = jnp.maximum(best, sc_)
                idx = jnp.where(sd_ > best, jnp.float32(3.0), idx)
                outbuf[i_out, pl.ds(jv * jnp.int32(16), 16)] = idx
                return carry_j

            return lax.fori_loop(0, pw // 16, jvec, carry_r)

        lax.fori_loop(0, _SC_R, row, jnp.int32(0))
        pltpu.sync_copy(outbuf,
                        o_hbm.at[chl, pl.ds(chunk * jnp.int32(_SC_R), _SC_R)])
        return carry

    lax.fori_loop(0, cpw * nchunk, unit, jnp.int32(0))


def kernel(x):
    batch, chan, h, w = x.shape
    nch = batch * chan
    ph, pw = h // 2, w // 2
    tc_ch = nch - _SC_CH
    cb = 4
    xr = x.reshape(nch, h, w)

    out_tc = pl.pallas_call(
        functools.partial(_mc_pool_tc, cb=cb, h=h, w=w),
        grid=(tc_ch // cb,),
        in_specs=[pl.BlockSpec((cb, h, w), lambda i: (i, 0, 0))],
        out_specs=pl.BlockSpec((cb, ph, pw), lambda i: (i, 0, 0)),
        out_shape=jax.ShapeDtypeStruct((tc_ch, ph, pw), jnp.float32),
        compiler_params=pltpu.CompilerParams(
            dimension_semantics=("parallel",)),
    )(xr)

    sc_fn = pl.kernel(
        functools.partial(_mc_pool_sc, ch_start=tc_ch, h=h, w=w),
        out_type=jax.ShapeDtypeStruct((_SC_CH, ph, pw), jnp.float32),
        mesh=plsc.VectorSubcoreMesh(core_axis_name="c", subcore_axis_name="s"),
        scratch_types=[
            pltpu.VMEM((2 * _SC_R, w), jnp.float32),
            pltpu.VMEM((_SC_R, pw), jnp.float32),
        ],
        compiler_params=pltpu.CompilerParams(needs_layout_passes=False),
    )
    out_sc = sc_fn(xr)

    out = jnp.concatenate([out_tc, out_sc], axis=0)
    return out.reshape(batch, chan, ph, pw)


# speedup vs baseline: 2.2155x; 1.0946x over previous
import functools

import jax
import jax.numpy as jnp
from jax import lax
from jax.experimental import pallas as pl
from jax.experimental.pallas import tpu as pltpu
from jax.experimental.pallas import tpu_sc as plsc

_TINY = float(jnp.finfo(jnp.float32).tiny)
_KS1 = 42
_KS2 = 0x1BD11BDA ^ 0 ^ 42
_ROT = ((13, 15, 26, 6), (17, 29, 16, 24))

_SC_CH = 88
_SC_R = 48


def _i32(v):
    return jnp.int32(jnp.uint32(v))


def _rotl(x, r):
    return (lax.shift_left(x, jnp.int32(r))
            | lax.shift_right_logical(x, jnp.int32(32 - r)))


def _threefry_bits(f):
    ks = (_i32(0), _i32(_KS1), _i32(_KS2))
    x1 = f + ks[1]
    x0 = x1
    x1 = lax.bitwise_xor(_rotl(x1, _ROT[0][0]), x0)
    for r in _ROT[0][1:]:
        x0 = x0 + x1
        x1 = lax.bitwise_xor(_rotl(x1, r), x0)
    x0 = x0 + ks[1]
    x1 = x1 + ks[2] + _i32(1)
    for i in range(1, 5):
        for r in _ROT[i % 2]:
            x0 = x0 + x1
            x1 = lax.bitwise_xor(_rotl(x1, r), x0)
        x0 = x0 + ks[(i + 1) % 3]
        x1 = x1 + ks[(i + 2) % 3] + _i32(i + 1)
    return lax.bitwise_xor(x0, x1)


def _uniform(f):
    bits = _threefry_bits(f)
    fb = lax.bitwise_or(lax.shift_right_logical(bits, jnp.int32(9)),
                        jnp.int32(0x3F800000))
    u = lax.bitcast_convert_type(fb, jnp.float32) - jnp.float32(1.0)
    return jnp.maximum(u, jnp.float32(_TINY))


def _score_tc(v, f):
    return v / (-jnp.log(_uniform(f)))


def _mc_pool_tc(x_ref, o_ref, *, cb, h, w):
    ch0 = pl.program_id(0) * cb
    ph, pw = h // 2, w // 2
    v = x_ref[...]

    ch = (lax.broadcasted_iota(jnp.int32, (cb, h, w), 0) + ch0) * _i32(h * w)
    r = lax.broadcasted_iota(jnp.int32, (cb, h, w), 1)
    c = lax.broadcasted_iota(jnp.int32, (cb, h, w), 2)
    f = (ch + r * _i32(2 * pw)
         - lax.bitwise_and(r, jnp.int32(1)) * _i32(2 * (pw - 1))
         + c * _i32(2) - lax.bitwise_and(c, jnp.int32(1)))

    s = _score_tc(v, f)
    sd = pltpu.roll(s, h - 1, 1)
    sr = pltpu.roll(s, w - 1, 2)
    sdr = pltpu.roll(sd, w - 1, 2)

    best = s
    idx = jnp.zeros_like(s)
    idx = jnp.where(sr > best, jnp.float32(1.0), idx)
    best = jnp.maximum(best, sr)
    idx = jnp.where(sd > best, jnp.float32(2.0), idx)
    best = jnp.maximum(best, sd)
    idx = jnp.where(sdr > best, jnp.float32(3.0), idx)

    csel = (lax.broadcasted_iota(jnp.int32, (w, pw), 0)
            == lax.broadcasted_iota(jnp.int32, (w, pw), 1) * 2
            ).astype(jnp.float32)
    rsel = (lax.broadcasted_iota(jnp.int32, (ph, h), 1)
            == lax.broadcasted_iota(jnp.int32, (ph, h), 0) * 2
            ).astype(jnp.float32)
    for b in range(cb):
        o_ref[b] = jnp.dot(rsel, jnp.dot(idx[b], csel))


def _neglog_sc(u):
    ub = lax.bitcast_convert_type(u, jnp.int32)
    e2 = lax.shift_right_logical(ub, jnp.int32(23)) - jnp.int32(127)
    m = lax.bitcast_convert_type(
        lax.bitwise_or(lax.bitwise_and(ub, jnp.int32(0x007FFFFF)),
                       jnp.int32(0x3F800000)), jnp.float32)
    big = m > jnp.float32(1.41421356)
    m = jnp.where(big, m * jnp.float32(0.5), m)
    e2 = jnp.where(big, e2 + jnp.int32(1), e2).astype(jnp.float32)
    x = m - jnp.float32(1.0)
    z = x * x
    p = jnp.float32(7.0376836292e-2)
    for c in (-1.1514610310e-1, 1.1676998740e-1, -1.2420140846e-1,
              1.4249322787e-1, -1.6668057665e-1, 2.0000714765e-1,
              -2.4999993993e-1, 3.3333331174e-1):
        p = p * x + jnp.float32(c)
    y = x * z * p
    y = y + e2 * jnp.float32(-2.12194440e-4)
    y = y - jnp.float32(0.5) * z
    r = x + y
    r = r + e2 * jnp.float32(0.693359375)
    return -r


def _mc_pool_sc(x_hbm, o_hbm, inbuf, outbuf, *, ch_start, h, w):
    ph, pw = h // 2, w // 2
    nw = 32
    nchunk = ph // _SC_R
    upw = _SC_CH * nchunk // nw
    wid = lax.axis_index("s") * jnp.int32(2) + lax.axis_index("c")
    g_lo = wid * jnp.int32(upw)
    ii = lax.iota(jnp.int32, 16)

    def unit(u, carry):
        g = g_lo + u
        chl = lax.shift_right_logical(g, jnp.int32(2))
        chunk = lax.bitwise_and(g, jnp.int32(3))
        pltpu.sync_copy(
            x_hbm.at[jnp.int32(ch_start) + chl,
                     pl.ds(chunk * jnp.int32(2 * _SC_R), 2 * _SC_R)],
            inbuf)

        def row(i_out, carry_r):
            r0 = i_out * jnp.int32(2)
            rv0 = jnp.broadcast_to(r0, (16,))
            rv1 = rv0 + jnp.int32(1)
            frow = ((jnp.int32(ch_start) + chl) * jnp.int32(h * w)
                    + (chunk * jnp.int32(_SC_R) + i_out) * jnp.int32(4 * pw))

            def jvec(jv, carry_j):
                ce = jv * jnp.int32(32) + ii * jnp.int32(2)
                co = ce + jnp.int32(1)
                a = plsc.load_gather(inbuf, [rv0, ce])
                b = plsc.load_gather(inbuf, [rv0, co])
                cc = plsc.load_gather(inbuf, [rv1, ce])
                d = plsc.load_gather(inbuf, [rv1, co])
                fv = frow + jv * jnp.int32(64) + ii * jnp.int32(4)
                sa = a / _neglog_sc(_uniform(fv))
                sb = b / _neglog_sc(_uniform(fv + jnp.int32(1)))
                sc_ = cc / _neglog_sc(_uniform(fv + jnp.int32(2)))
                sd_ = d / _neglog_sc(_uniform(fv + jnp.int32(3)))
                best = sa
                idx = jnp.zeros_like(sa)
                idx = jnp.where(sb > best, jnp.float32(1.0), idx)
                best = jnp.maximum(best, sb)
                idx = jnp.where(sc_ > best, jnp.float32(2.0), idx)
                best = jnp.maximum(best, sc_)
                idx = jnp.where(sd_ > best, jnp.float32(3.0), idx)
                outbuf[i_out, pl.ds(jv * jnp.int32(16), 16)] = idx
                return carry_j

            return lax.fori_loop(0, pw // 16, jvec, carry_r)

        lax.fori_loop(0, _SC_R, row, jnp.int32(0))
        pltpu.sync_copy(outbuf,
                        o_hbm.at[chl, pl.ds(chunk * jnp.int32(_SC_R), _SC_R)])
        return carry

    lax.fori_loop(0, upw, unit, jnp.int32(0))


def kernel(x):
    batch, chan, h, w = x.shape
    nch = batch * chan
    ph, pw = h // 2, w // 2
    tc_ch = nch - _SC_CH
    cb = 4
    xr = x.reshape(nch, h, w)

    out_tc = pl.pallas_call(
        functools.partial(_mc_pool_tc, cb=cb, h=h, w=w),
        grid=(tc_ch // cb,),
        in_specs=[pl.BlockSpec((cb, h, w), lambda i: (i, 0, 0))],
        out_specs=pl.BlockSpec((cb, ph, pw), lambda i: (i, 0, 0)),
        out_shape=jax.ShapeDtypeStruct((nch, ph, pw), jnp.float32),
        compiler_params=pltpu.CompilerParams(
            dimension_semantics=("parallel",)),
    )(xr)

    sc_fn = pl.kernel(
        functools.partial(_mc_pool_sc, ch_start=tc_ch, h=h, w=w),
        out_type=jax.ShapeDtypeStruct((_SC_CH, ph, pw), jnp.float32),
        mesh=plsc.VectorSubcoreMesh(core_axis_name="c", subcore_axis_name="s"),
        scratch_types=[
            pltpu.VMEM((2 * _SC_R, w), jnp.float32),
            pltpu.VMEM((_SC_R, pw), jnp.float32),
        ],
        compiler_params=pltpu.CompilerParams(needs_layout_passes=False),
    )
    out_sc = sc_fn(xr)

    out = lax.dynamic_update_slice(out_tc, out_sc, (tc_ch, 0, 0))
    return out.reshape(batch, chan, ph, pw)


# speedup vs baseline: 2.2675x; 1.0235x over previous
import functools

import jax
import jax.numpy as jnp
import numpy as np
from jax import lax
from jax.experimental import pallas as pl
from jax.experimental.pallas import tpu as pltpu
from jax.experimental.pallas import tpu_sc as plsc

_TINY = float(jnp.finfo(jnp.float32).tiny)
_KS1 = 42
_KS2 = 0x1BD11BDA ^ 0 ^ 42
_ROT = ((13, 15, 26, 6), (17, 29, 16, 24))

_SC_CH = 88
_SC_R = 48


def _i32(v):
    return jnp.int32(jnp.uint32(v))


def _rotl(x, r):
    return (lax.shift_left(x, jnp.int32(r))
            | lax.shift_right_logical(x, jnp.int32(32 - r)))


def _threefry_bits(x1):
    ks = (_i32(0), _i32(_KS1), _i32(_KS2))
    x0 = x1
    x1 = lax.bitwise_xor(_rotl(x1, _ROT[0][0]), x0)
    for r in _ROT[0][1:]:
        x0 = x0 + x1
        x1 = lax.bitwise_xor(_rotl(x1, r), x0)
    x0 = x0 + ks[1]
    x1 = x1 + ks[2] + _i32(1)
    for i in range(1, 5):
        for r in _ROT[i % 2]:
            x0 = x0 + x1
            x1 = lax.bitwise_xor(_rotl(x1, r), x0)
        x0 = x0 + ks[(i + 1) % 3]
        x1 = x1 + ks[(i + 2) % 3] + _i32(i + 1)
    return lax.bitwise_xor(x0, x1)


def _uniform(x1):
    bits = _threefry_bits(x1)
    fb = lax.bitwise_or(lax.shift_right_logical(bits, jnp.int32(9)),
                        jnp.int32(0x3F800000))
    u = lax.bitcast_convert_type(fb, jnp.float32) - jnp.float32(1.0)
    return jnp.maximum(u, jnp.float32(_TINY))


def _score_tc(v, f):
    return v / (-jnp.log(_uniform(f)))


def _mc_pool_tc(x_ref, fb_ref, o_ref, *, cb, h, w):
    ch0 = pl.program_id(0) * cb
    ph, pw = h // 2, w // 2
    v = x_ref[...]

    f = fb_ref[...] + ch0 * _i32(h * w)

    s = _score_tc(v, f)
    sd = pltpu.roll(s, h - 1, 1)
    sr = pltpu.roll(s, w - 1, 2)
    sdr = pltpu.roll(sd, w - 1, 2)

    best = s
    idx = jnp.zeros_like(s)
    idx = jnp.where(sr > best, jnp.float32(1.0), idx)
    best = jnp.maximum(best, sr)
    idx = jnp.where(sd > best, jnp.float32(2.0), idx)
    best = jnp.maximum(best, sd)
    idx = jnp.where(sdr > best, jnp.float32(3.0), idx)

    csel = (lax.broadcasted_iota(jnp.int32, (w, pw), 0)
            == lax.broadcasted_iota(jnp.int32, (w, pw), 1) * 2
            ).astype(jnp.float32)
    rsel = (lax.broadcasted_iota(jnp.int32, (ph, h), 1)
            == lax.broadcasted_iota(jnp.int32, (ph, h), 0) * 2
            ).astype(jnp.float32)
    for b in range(cb):
        o_ref[b] = jnp.dot(rsel, jnp.dot(idx[b], csel))


def _neglog_sc(u):
    ub = lax.bitcast_convert_type(u, jnp.int32)
    e2 = lax.shift_right_logical(ub, jnp.int32(23)) - jnp.int32(127)
    m = lax.bitcast_convert_type(
        lax.bitwise_or(lax.bitwise_and(ub, jnp.int32(0x007FFFFF)),
                       jnp.int32(0x3F800000)), jnp.float32)
    big = m > jnp.float32(1.41421356)
    m = jnp.where(big, m * jnp.float32(0.5), m)
    e2 = jnp.where(big, e2 + jnp.int32(1), e2).astype(jnp.float32)
    x = m - jnp.float32(1.0)
    z = x * x
    p = jnp.float32(7.0376836292e-2)
    for c in (-1.1514610310e-1, 1.1676998740e-1, -1.2420140846e-1,
              1.4249322787e-1, -1.6668057665e-1, 2.0000714765e-1,
              -2.4999993993e-1, 3.3333331174e-1):
        p = p * x + jnp.float32(c)
    y = x * z * p
    y = y + e2 * jnp.float32(-2.12194440e-4)
    y = y - jnp.float32(0.5) * z
    r = x + y
    r = r + e2 * jnp.float32(0.693359375)
    return -r


def _mc_pool_sc(x_hbm, o_hbm, inbuf, outbuf, *, ch_start, h, w):
    ph, pw = h // 2, w // 2
    nw = 32
    nchunk = ph // _SC_R
    upw = _SC_CH * nchunk // nw
    wid = lax.axis_index("s") * jnp.int32(2) + lax.axis_index("c")
    g_lo = wid * jnp.int32(upw)
    ii = lax.iota(jnp.int32, 16)

    def unit(u, carry):
        g = g_lo + u
        chl = lax.shift_right_logical(g, jnp.int32(2))
        chunk = lax.bitwise_and(g, jnp.int32(3))
        pltpu.sync_copy(
            x_hbm.at[jnp.int32(ch_start) + chl,
                     pl.ds(chunk * jnp.int32(2 * _SC_R), 2 * _SC_R)],
            inbuf)

        def row(i_out, carry_r):
            r0 = i_out * jnp.int32(2)
            rv0 = jnp.broadcast_to(r0, (16,))
            rv1 = rv0 + jnp.int32(1)
            frow = ((jnp.int32(ch_start) + chl) * jnp.int32(h * w)
                    + (chunk * jnp.int32(_SC_R) + i_out) * jnp.int32(4 * pw)
                    + jnp.int32(42))

            def jvec(jv, carry_j):
                ce = jv * jnp.int32(32) + ii * jnp.int32(2)
                co = ce + jnp.int32(1)
                a = plsc.load_gather(inbuf, [rv0, ce])
                b = plsc.load_gather(inbuf, [rv0, co])
                cc = plsc.load_gather(inbuf, [rv1, ce])
                d = plsc.load_gather(inbuf, [rv1, co])
                fv = frow + jv * jnp.int32(64) + ii * jnp.int32(4)
                sa = a / _neglog_sc(_uniform(fv))
                sb = b / _neglog_sc(_uniform(fv + jnp.int32(1)))
                sc_ = cc / _neglog_sc(_uniform(fv + jnp.int32(2)))
                sd_ = d / _neglog_sc(_uniform(fv + jnp.int32(3)))
                best = sa
                idx = jnp.zeros_like(sa)
                idx = jnp.where(sb > best, jnp.float32(1.0), idx)
                best = jnp.maximum(best, sb)
                idx = jnp.where(sc_ > best, jnp.float32(2.0), idx)
                best = jnp.maximum(best, sc_)
                idx = jnp.where(sd_ > best, jnp.float32(3.0), idx)
                outbuf[i_out, pl.ds(jv * jnp.int32(16), 16)] = idx
                return carry_j

            return lax.fori_loop(0, pw // 16, jvec, carry_r)

        lax.fori_loop(0, _SC_R, row, jnp.int32(0))
        pltpu.sync_copy(outbuf,
                        o_hbm.at[chl, pl.ds(chunk * jnp.int32(_SC_R), _SC_R)])
        return carry

    lax.fori_loop(0, upw, unit, jnp.int32(0))


def kernel(x):
    batch, chan, h, w = x.shape
    nch = batch * chan
    ph, pw = h // 2, w // 2
    tc_ch = nch - _SC_CH
    cb = 4
    xr = x.reshape(nch, h, w)

    chn = np.arange(cb, dtype=np.int64)[:, None, None]
    rn = np.arange(h, dtype=np.int64)[None, :, None]
    cn = np.arange(w, dtype=np.int64)[None, None, :]
    fb_np = (chn * (h * w) + rn * w - (rn & 1) * (w - 2)
             + 2 * cn - (cn & 1) + 42)
    fb = jnp.asarray(fb_np.astype(np.int64).astype(np.uint32).view(np.int32))

    out_tc = pl.pallas_call(
        functools.partial(_mc_pool_tc, cb=cb, h=h, w=w),
        grid=(tc_ch // cb,),
        in_specs=[
            pl.BlockSpec((cb, h, w), lambda i: (i, 0, 0)),
            pl.BlockSpec((cb, h, w), lambda i: (0, 0, 0)),
        ],
        out_specs=pl.BlockSpec((cb, ph, pw), lambda i: (i, 0, 0)),
        out_shape=jax.ShapeDtypeStruct((nch, ph, pw), jnp.float32),
        compiler_params=pltpu.CompilerParams(
            dimension_semantics=("parallel",)),
    )(xr, fb)

    sc_fn = pl.kernel(
        functools.partial(_mc_pool_sc, ch_start=tc_ch, h=h, w=w),
        out_type=jax.ShapeDtypeStruct((_SC_CH, ph, pw), jnp.float32),
        mesh=plsc.VectorSubcoreMesh(core_axis_name="c", subcore_axis_name="s"),
        scratch_types=[
            pltpu.VMEM((2 * _SC_R, w), jnp.float32),
            pltpu.VMEM((_SC_R, pw), jnp.float32),
        ],
        compiler_params=pltpu.CompilerParams(needs_layout_passes=False),
    )
    out_sc = sc_fn(xr)

    out = lax.dynamic_update_slice(out_tc, out_sc, (tc_ch, 0, 0))
    return out.reshape(batch, chan, ph, pw)


# speedup vs baseline: 2.2869x; 1.0086x over previous
import functools

import jax
import jax.numpy as jnp
import numpy as np
from jax import lax
from jax.experimental import pallas as pl
from jax.experimental.pallas import tpu as pltpu
from jax.experimental.pallas import tpu_sc as plsc

_TINY = float(jnp.finfo(jnp.float32).tiny)
_KS1 = 42
_KS2 = 0x1BD11BDA ^ 0 ^ 42
_ROT = ((13, 15, 26, 6), (17, 29, 16, 24))

_SC_CH = 88
_SC_R = 48


def _i32(v):
    return jnp.int32(jnp.uint32(v))


def _rotl(x, r):
    return (lax.shift_left(x, jnp.int32(r))
            | lax.shift_right_logical(x, jnp.int32(32 - r)))


def _threefry_bits(x1):
    ks = (_i32(0), _i32(_KS1), _i32(_KS2))
    x0 = x1
    x1 = lax.bitwise_xor(_rotl(x1, _ROT[0][0]), x0)
    for r in _ROT[0][1:]:
        x0 = x0 + x1
        x1 = lax.bitwise_xor(_rotl(x1, r), x0)
    x0 = x0 + ks[1]
    x1 = x1 + ks[2] + _i32(1)
    for i in range(1, 5):
        for r in _ROT[i % 2]:
            x0 = x0 + x1
            x1 = lax.bitwise_xor(_rotl(x1, r), x0)
        x0 = x0 + ks[(i + 1) % 3]
        x1 = x1 + ks[(i + 2) % 3] + _i32(i + 1)
    return lax.bitwise_xor(x0, x1)


def _uniform(x1):
    bits = _threefry_bits(x1)
    fb = lax.bitwise_or(lax.shift_right_logical(bits, jnp.int32(9)),
                        jnp.int32(0x3F800000))
    u = lax.bitcast_convert_type(fb, jnp.float32) - jnp.float32(1.0)
    return jnp.maximum(u, jnp.float32(_TINY))


def _score_tc(v, f):
    return v / (-jnp.log(_uniform(f)))


def _mc_pool_tc(x_ref, fb_ref, o_ref, *, cb, h, w):
    ch0 = pl.program_id(0) * cb
    ph, pw = h // 2, w // 2
    v = x_ref[...]

    f = fb_ref[...] + ch0 * _i32(h * w)

    s = _score_tc(v, f)
    sd = pltpu.roll(s, h - 1, 1)
    sr = pltpu.roll(s, w - 1, 2)
    sdr = pltpu.roll(sd, w - 1, 2)

    best = s
    idx = jnp.zeros_like(s)
    idx = jnp.where(sr > best, jnp.float32(1.0), idx)
    best = jnp.maximum(best, sr)
    idx = jnp.where(sd > best, jnp.float32(2.0), idx)
    best = jnp.maximum(best, sd)
    idx = jnp.where(sdr > best, jnp.float32(3.0), idx)

    csel = (lax.broadcasted_iota(jnp.int32, (w, pw), 0)
            == lax.broadcasted_iota(jnp.int32, (w, pw), 1) * 2
            ).astype(jnp.float32)
    rsel = (lax.broadcasted_iota(jnp.int32, (ph, h), 1)
            == lax.broadcasted_iota(jnp.int32, (ph, h), 0) * 2
            ).astype(jnp.float32)
    for b in range(cb):
        o_ref[b] = jnp.dot(rsel, jnp.dot(idx[b], csel))


def _neglog_sc(u):
    ub = lax.bitcast_convert_type(u, jnp.int32)
    e2 = lax.shift_right_logical(ub, jnp.int32(23)) - jnp.int32(127)
    m = lax.bitcast_convert_type(
        lax.bitwise_or(lax.bitwise_and(ub, jnp.int32(0x007FFFFF)),
                       jnp.int32(0x3F800000)), jnp.float32)
    big = m > jnp.float32(1.41421356)
    m = jnp.where(big, m * jnp.float32(0.5), m)
    e2 = jnp.where(big, e2 + jnp.int32(1), e2).astype(jnp.float32)
    x = m - jnp.float32(1.0)
    z = x * x
    p = jnp.float32(7.0376836292e-2)
    for c in (-1.1514610310e-1, 1.1676998740e-1, -1.2420140846e-1,
              1.4249322787e-1, -1.6668057665e-1, 2.0000714765e-1,
              -2.4999993993e-1, 3.3333331174e-1):
        p = p * x + jnp.float32(c)
    y = x * z * p
    y = y + e2 * jnp.float32(-2.12194440e-4)
    y = y - jnp.float32(0.5) * z
    r = x + y
    r = r + e2 * jnp.float32(0.693359375)
    return -r


def _mc_pool_sc(x_hbm, o_hbm, inbuf, outbuf, *, ch_start, h, w):
    ph, pw = h // 2, w // 2
    nw = 32
    nchunk = ph // _SC_R
    upw = _SC_CH * nchunk // nw
    wid = lax.axis_index("s") * jnp.int32(2) + lax.axis_index("c")
    g_lo = wid * jnp.int32(upw)
    ii = lax.iota(jnp.int32, 16)

    def unit(u, carry):
        g = g_lo + u
        chl = lax.shift_right_logical(g, jnp.int32(2))
        chunk = lax.bitwise_and(g, jnp.int32(3))
        pltpu.sync_copy(
            x_hbm.at[jnp.int32(ch_start) + chl,
                     pl.ds(chunk * jnp.int32(2 * _SC_R), 2 * _SC_R)],
            inbuf)

        def row(i_out, carry_r):
            r0 = i_out * jnp.int32(2)
            rv0 = jnp.broadcast_to(r0, (16,))
            rv1 = rv0 + jnp.int32(1)
            frow = ((jnp.int32(ch_start) + chl) * jnp.int32(h * w)
                    + (chunk * jnp.int32(_SC_R) + i_out) * jnp.int32(4 * pw)
                    + jnp.int32(42))

            def jvec(jv, carry_j):
                ce = jv * jnp.int32(32) + ii * jnp.int32(2)
                co = ce + jnp.int32(1)
                a = plsc.load_gather(inbuf, [rv0, ce])
                b = plsc.load_gather(inbuf, [rv0, co])
                cc = plsc.load_gather(inbuf, [rv1, ce])
                d = plsc.load_gather(inbuf, [rv1, co])
                fv = frow + jv * jnp.int32(64) + ii * jnp.int32(4)
                sa = a / _neglog_sc(_uniform(fv))
                sb = b / _neglog_sc(_uniform(fv + jnp.int32(1)))
                sc_ = cc / _neglog_sc(_uniform(fv + jnp.int32(2)))
                sd_ = d / _neglog_sc(_uniform(fv + jnp.int32(3)))
                best = sa
                idx = jnp.zeros_like(sa)
                idx = jnp.where(sb > best, jnp.float32(1.0), idx)
                best = jnp.maximum(best, sb)
                idx = jnp.where(sc_ > best, jnp.float32(2.0), idx)
                best = jnp.maximum(best, sc_)
                idx = jnp.where(sd_ > best, jnp.float32(3.0), idx)
                outbuf[i_out, pl.ds(jv * jnp.int32(16), 16)] = idx
                return carry_j

            return lax.fori_loop(0, pw // 16, jvec, carry_r)

        lax.fori_loop(0, _SC_R, row, jnp.int32(0))
        pltpu.sync_copy(outbuf,
                        o_hbm.at[chl, pl.ds(chunk * jnp.int32(_SC_R), _SC_R)])
        return carry

    lax.fori_loop(0, upw, unit, jnp.int32(0))


def kernel(x):
    batch, chan, h, w = x.shape
    nch = batch * chan
    ph, pw = h // 2, w // 2
    tc_ch = nch - _SC_CH
    cb = 8
    xr = x.reshape(nch, h, w)

    chn = np.arange(cb, dtype=np.int64)[:, None, None]
    rn = np.arange(h, dtype=np.int64)[None, :, None]
    cn = np.arange(w, dtype=np.int64)[None, None, :]
    fb_np = (chn * (h * w) + rn * w - (rn & 1) * (w - 2)
             + 2 * cn - (cn & 1) + 42)
    fb = jnp.asarray(fb_np.astype(np.int64).astype(np.uint32).view(np.int32))

    out_tc = pl.pallas_call(
        functools.partial(_mc_pool_tc, cb=cb, h=h, w=w),
        grid=(tc_ch // cb,),
        in_specs=[
            pl.BlockSpec((cb, h, w), lambda i: (i, 0, 0)),
            pl.BlockSpec((cb, h, w), lambda i: (0, 0, 0)),
        ],
        out_specs=pl.BlockSpec((cb, ph, pw), lambda i: (i, 0, 0)),
        out_shape=jax.ShapeDtypeStruct((nch, ph, pw), jnp.float32),
        compiler_params=pltpu.CompilerParams(
            dimension_semantics=("parallel",)),
    )(xr, fb)

    sc_fn = pl.kernel(
        functools.partial(_mc_pool_sc, ch_start=tc_ch, h=h, w=w),
        out_type=jax.ShapeDtypeStruct((_SC_CH, ph, pw), jnp.float32),
        mesh=plsc.VectorSubcoreMesh(core_axis_name="c", subcore_axis_name="s"),
        scratch_types=[
            pltpu.VMEM((2 * _SC_R, w), jnp.float32),
            pltpu.VMEM((_SC_R, pw), jnp.float32),
        ],
        compiler_params=pltpu.CompilerParams(needs_layout_passes=False),
    )
    out_sc = sc_fn(xr)

    out = lax.dynamic_update_slice(out_tc, out_sc, (tc_ch, 0, 0))
    return out.reshape(batch, chan, ph, pw)


# speedup vs baseline: 2.2940x; 1.0031x over previous
import functools

import jax
import jax.numpy as jnp
import numpy as np
from jax import lax
from jax.experimental import pallas as pl
from jax.experimental.pallas import tpu as pltpu
from jax.experimental.pallas import tpu_sc as plsc

_TINY = float(jnp.finfo(jnp.float32).tiny)
_KS1 = 42
_KS2 = 0x1BD11BDA ^ 0 ^ 42
_ROT = ((13, 15, 26, 6), (17, 29, 16, 24))

_SC_CH = 88
_SC_R = 48


def _i32(v):
    return jnp.int32(jnp.uint32(v))


def _rotl(x, r):
    return (lax.shift_left(x, jnp.int32(r))
            | lax.shift_right_logical(x, jnp.int32(32 - r)))


def _threefry_bits(x1):
    ks = (_i32(0), _i32(_KS1), _i32(_KS2))
    x0 = x1
    x1 = lax.bitwise_xor(_rotl(x1, _ROT[0][0]), x0)
    for r in _ROT[0][1:]:
        x0 = x0 + x1
        x1 = lax.bitwise_xor(_rotl(x1, r), x0)
    x0 = x0 + ks[1]
    x1 = x1 + ks[2] + _i32(1)
    for i in range(1, 5):
        for r in _ROT[i % 2]:
            x0 = x0 + x1
            x1 = lax.bitwise_xor(_rotl(x1, r), x0)
        x0 = x0 + ks[(i + 1) % 3]
        x1 = x1 + ks[(i + 2) % 3] + _i32(i + 1)
    return lax.bitwise_xor(x0, x1)


def _uniform(x1):
    bits = _threefry_bits(x1)
    fb = lax.bitwise_or(lax.shift_right_logical(bits, jnp.int32(9)),
                        jnp.int32(0x3F800000))
    u = lax.bitcast_convert_type(fb, jnp.float32) - jnp.float32(1.0)
    return jnp.maximum(u, jnp.float32(_TINY))


def _score_tc(v, f):
    return v / (-jnp.log(_uniform(f)))


def _mc_pool_tc(x_ref, fb_ref, o_ref, *, cb, h, w):
    ch0 = pl.program_id(0) * cb
    ph, pw = h // 2, w // 2
    v = x_ref[...]

    f = fb_ref[...] + ch0 * _i32(h * w)

    s = _score_tc(v, f)
    sd = pltpu.roll(s, h - 1, 1)
    sr = pltpu.roll(s, w - 1, 2)
    sdr = pltpu.roll(sd, w - 1, 2)

    best = s
    idx = jnp.zeros_like(s)
    idx = jnp.where(sr > best, jnp.float32(1.0), idx)
    best = jnp.maximum(best, sr)
    idx = jnp.where(sd > best, jnp.float32(2.0), idx)
    best = jnp.maximum(best, sd)
    idx = jnp.where(sdr > best, jnp.float32(3.0), idx)

    csel = (lax.broadcasted_iota(jnp.int32, (w, pw), 0)
            == lax.broadcasted_iota(jnp.int32, (w, pw), 1) * 2
            ).astype(jnp.float32)
    rsel = (lax.broadcasted_iota(jnp.int32, (ph, h), 1)
            == lax.broadcasted_iota(jnp.int32, (ph, h), 0) * 2
            ).astype(jnp.float32)
    for b in range(cb):
        o_ref[b] = jnp.dot(rsel, jnp.dot(idx[b], csel))


def _neglog_sc(u):
    ub = lax.bitcast_convert_type(u, jnp.int32)
    e2 = lax.shift_right_logical(ub, jnp.int32(23)) - jnp.int32(127)
    m = lax.bitcast_convert_type(
        lax.bitwise_or(lax.bitwise_and(ub, jnp.int32(0x007FFFFF)),
                       jnp.int32(0x3F800000)), jnp.float32)
    big = m > jnp.float32(1.41421356)
    m = jnp.where(big, m * jnp.float32(0.5), m)
    e2 = jnp.where(big, e2 + jnp.int32(1), e2).astype(jnp.float32)
    x = m - jnp.float32(1.0)
    z = x * x
    p = jnp.float32(7.0376836292e-2)
    for c in (-1.1514610310e-1, 1.1676998740e-1, -1.2420140846e-1,
              1.4249322787e-1, -1.6668057665e-1, 2.0000714765e-1,
              -2.4999993993e-1, 3.3333331174e-1):
        p = p * x + jnp.float32(c)
    y = x * z * p
    y = y + e2 * jnp.float32(-2.12194440e-4)
    y = y - jnp.float32(0.5) * z
    r = x + y
    r = r + e2 * jnp.float32(0.693359375)
    return -r


def _mc_pool_sc(x_hbm, o_hbm, inbuf0, inbuf1, outbuf, sem0, sem1,
                *, ch_start, h, w):
    ph, pw = h // 2, w // 2
    nw = 32
    nchunk = ph // _SC_R
    upw = _SC_CH * nchunk // nw
    wid = lax.axis_index("s") * jnp.int32(2) + lax.axis_index("c")
    g_lo = wid * jnp.int32(upw)
    g_hi = g_lo + jnp.int32(upw - 1)
    ii = lax.iota(jnp.int32, 16)

    def in_copy(g, buf, sem):
        g = jnp.minimum(g, g_hi)
        chl = lax.shift_right_logical(g, jnp.int32(2))
        chunk = lax.bitwise_and(g, jnp.int32(3))
        return pltpu.make_async_copy(
            x_hbm.at[jnp.int32(ch_start) + chl,
                     pl.ds(chunk * jnp.int32(2 * _SC_R), 2 * _SC_R)],
            buf, sem)

    def compute(g, inbuf):
        chl = lax.shift_right_logical(g, jnp.int32(2))
        chunk = lax.bitwise_and(g, jnp.int32(3))

        def row(i_out, carry_r):
            r0 = i_out * jnp.int32(2)
            rv0 = jnp.broadcast_to(r0, (16,))
            rv1 = rv0 + jnp.int32(1)
            frow = ((jnp.int32(ch_start) + chl) * jnp.int32(h * w)
                    + (chunk * jnp.int32(_SC_R) + i_out) * jnp.int32(4 * pw)
                    + jnp.int32(42))

            def jvec(jv, carry_j):
                ce = jv * jnp.int32(32) + ii * jnp.int32(2)
                co = ce + jnp.int32(1)
                a = plsc.load_gather(inbuf, [rv0, ce])
                b = plsc.load_gather(inbuf, [rv0, co])
                cc = plsc.load_gather(inbuf, [rv1, ce])
                d = plsc.load_gather(inbuf, [rv1, co])
                fv = frow + jv * jnp.int32(64) + ii * jnp.int32(4)
                sa = a / _neglog_sc(_uniform(fv))
                sb = b / _neglog_sc(_uniform(fv + jnp.int32(1)))
                sc_ = cc / _neglog_sc(_uniform(fv + jnp.int32(2)))
                sd_ = d / _neglog_sc(_uniform(fv + jnp.int32(3)))
                best = sa
                idx = jnp.zeros_like(sa)
                idx = jnp.where(sb > best, jnp.float32(1.0), idx)
                best = jnp.maximum(best, sb)
                idx = jnp.where(sc_ > best, jnp.float32(2.0), idx)
                best = jnp.maximum(best, sc_)
                idx = jnp.where(sd_ > best, jnp.float32(3.0), idx)
                outbuf[i_out, pl.ds(jv * jnp.int32(16), 16)] = idx
                return carry_j

            return lax.fori_loop(0, pw // 16, jvec, carry_r)

        lax.fori_loop(0, _SC_R, row, jnp.int32(0))
        pltpu.sync_copy(outbuf,
                        o_hbm.at[chl, pl.ds(chunk * jnp.int32(_SC_R), _SC_R)])

    npairs = upw // 2
    has_tail = upw % 2 == 1
    in_copy(g_lo, inbuf0, sem0).start()
    in_copy(g_lo + jnp.int32(1), inbuf1, sem1).start()

    def pair(t, carry):
        g0 = g_lo + t * jnp.int32(2)
        in_copy(g0, inbuf0, sem0).wait()
        compute(g0, inbuf0)
        in_copy(g0 + jnp.int32(2), inbuf0, sem0).start()
        in_copy(g0 + jnp.int32(1), inbuf1, sem1).wait()
        compute(g0 + jnp.int32(1), inbuf1)
        in_copy(g0 + jnp.int32(3), inbuf1, sem1).start()
        return carry

    lax.fori_loop(0, npairs, pair, jnp.int32(0))
    g_tail = g_lo + jnp.int32(2 * npairs)
    in_copy(g_tail, inbuf0, sem0).wait()
    if has_tail:
        compute(g_tail, inbuf0)
    in_copy(g_tail, inbuf1, sem1).wait()


def kernel(x):
    batch, chan, h, w = x.shape
    nch = batch * chan
    ph, pw = h // 2, w // 2
    tc_ch = nch - _SC_CH
    cb = 8
    xr = x.reshape(nch, h, w)

    chn = np.arange(cb, dtype=np.int64)[:, None, None]
    rn = np.arange(h, dtype=np.int64)[None, :, None]
    cn = np.arange(w, dtype=np.int64)[None, None, :]
    fb_np = (chn * (h * w) + rn * w - (rn & 1) * (w - 2)
             + 2 * cn - (cn & 1) + 42)
    fb = jnp.asarray(fb_np.astype(np.int64).astype(np.uint32).view(np.int32))

    out_tc = pl.pallas_call(
        functools.partial(_mc_pool_tc, cb=cb, h=h, w=w),
        grid=(tc_ch // cb,),
        in_specs=[
            pl.BlockSpec((cb, h, w), lambda i: (i, 0, 0)),
            pl.BlockSpec((cb, h, w), lambda i: (0, 0, 0)),
        ],
        out_specs=pl.BlockSpec((cb, ph, pw), lambda i: (i, 0, 0)),
        out_shape=jax.ShapeDtypeStruct((nch, ph, pw), jnp.float32),
        compiler_params=pltpu.CompilerParams(
            dimension_semantics=("parallel",)),
    )(xr, fb)

    sc_fn = pl.kernel(
        functools.partial(_mc_pool_sc, ch_start=tc_ch, h=h, w=w),
        out_type=jax.ShapeDtypeStruct((_SC_CH, ph, pw), jnp.float32),
        mesh=plsc.VectorSubcoreMesh(core_axis_name="c", subcore_axis_name="s"),
        scratch_types=[
            pltpu.VMEM((2 * _SC_R, w), jnp.float32),
            pltpu.VMEM((2 * _SC_R, w), jnp.float32),
            pltpu.VMEM((_SC_R, pw), jnp.float32),
            pltpu.SemaphoreType.DMA,
            pltpu.SemaphoreType.DMA,
        ],
        compiler_params=pltpu.CompilerParams(needs_layout_passes=False),
    )
    out_sc = sc_fn(xr)

    out = lax.dynamic_update_slice(out_tc, out_sc, (tc_ch, 0, 0))
    return out.reshape(batch, chan, ph, pw)
